# baseline (speedup 1.0000x reference)
import functools

import jax
import jax.numpy as jnp
from jax import lax
from jax.experimental import pallas as pl
from jax.experimental.pallas import tpu as pltpu
from jax.experimental.pallas import tpu_sc as plsc

_BATCH = 4096
_QLEN = 20
_DIM = 16
_VOCAB = 1000000
_NC = 2
_NS = 16
_NW = _NC * _NS
_BPW = _BATCH // _NW
_G = _VOCAB // 8
_KB = 8192
_KG = 123

_mesh = plsc.VectorSubcoreMesh(core_axis_name="c", subcore_axis_name="s")


_TC = 7813
_CH = 8
_NCHK = 976
_GP = 125008


@jax.jit
def _repack(table_t):
    @functools.partial(
        pl.kernel,
        mesh=_mesh,
        out_type=jax.ShapeDtypeStruct((_GP, 128), jnp.float32),
        scratch_types=[
            pltpu.VMEM((2, _DIM, 128 * _CH), jnp.float32),
            pltpu.VMEM((2, 16 * _CH, 128), jnp.float32),
            pltpu.SemaphoreType.DMA,
            pltpu.SemaphoreType.DMA,
            pltpu.SemaphoreType.DMA,
            pltpu.SemaphoreType.DMA,
        ],
        compiler_params=pltpu.CompilerParams(
            use_tc_tiling_on_sc=True, needs_layout_passes=False
        ),
    )
    def body(tt_hbm, out_hbm, tin, tout, si0, si1, so0, so1):
        wid = lax.axis_index("s") * _NC + lax.axis_index("c")
        sis = (si0, si1)
        sos = (so0, so1)
        iota = lax.iota(jnp.int32, 16)

        def fire_in(j, buf):
            cid = wid + jnp.int32(32) * j
            pltpu.async_copy(
                tt_hbm.at[:, pl.ds(cid * (128 * _CH), 128 * _CH)],
                tin.at[buf],
                sis[buf],
            )

        def wait_in(buf):
            pltpu.make_async_copy(
                tt_hbm.at[:, pl.ds(0, 128 * _CH)], tin.at[buf], sis[buf]
            ).wait()

        def wait_out(buf):
            pltpu.make_async_copy(
                tt_hbm.at[:, pl.ds(0, 128 * _CH)], tout.at[buf], sos[buf]
            ).wait()

        def compute_store(j, buf):
            @pl.loop(0, _CH)
            def _(t):
                for g_in in range(16):
                    for a in range(8):
                        col = jnp.full(
                            (16,), t * 128 + jnp.int32(8 * g_in + a), jnp.int32
                        )
                        src = plsc.load_gather(tin.at[buf], [iota, col])
                        tout[buf, 16 * t + g_in, pl.ds(a * 16, 16)] = src

            cid = wid + jnp.int32(32) * j
            pltpu.async_copy(
                tout.at[buf], out_hbm.at[pl.ds(cid * (16 * _CH), 16 * _CH), :],
                sos[buf],
            )

        fire_in(jnp.int32(0), 0)

        @pl.loop(0, 32, step=2)
        def _(j):
            for b in range(2):
                jj = j + b

                @pl.when(wid + 32 * jj < _NCHK)
                def _(jj=jj, b=b):
                    @pl.when(wid + 32 * (jj + 1) < _NCHK)
                    def _():
                        fire_in(jj + 1, 1 - b)

                    wait_in(b)

                    @pl.when(jj >= 2)
                    def _():
                        wait_out(b)

                    compute_store(jj, b)

        @pl.when(wid < _TC - _CH * _NCHK)
        def _():
            tc = jnp.int32(_CH * _NCHK) + wid
            pltpu.async_copy(
                tt_hbm.at[:, pl.ds(tc * 128, 128)],
                tin.at[0, :, pl.ds(0, 128)],
                sis[0],
            )
            pltpu.make_async_copy(
                tt_hbm.at[:, pl.ds(0, 128)],
                tin.at[0, :, pl.ds(0, 128)],
                sis[0],
            ).wait()
            wait_out(0)
            for g_in in range(16):
                for a in range(8):
                    col = jnp.full((16,), 8 * g_in + a, jnp.int32)
                    src = plsc.load_gather(tin.at[0], [iota, col])
                    tout[0, g_in, pl.ds(a * 16, 16)] = src
            pltpu.sync_copy(
                tout.at[0, pl.ds(0, 16), :],
                out_hbm.at[pl.ds(tc * 16, 16), :],
            )

        @pl.when(wid >= _TC - _CH * _NCHK)
        def _():
            wait_out(0)

        wait_out(1)

    return body(table_t)


@jax.jit
def _gather(query_pad, table8):
    @functools.partial(
        pl.kernel,
        mesh=_mesh,
        out_type=jax.ShapeDtypeStruct((_QLEN, 2, 8 * _NW, _BPW), jnp.float32),
        scratch_types=[
            pltpu.VMEM((24, _BPW), jnp.int32),
            pltpu.VMEM((2, _BPW), jnp.int32),
            pltpu.VMEM((2, _BPW, 128), jnp.float32),
            pltpu.VMEM((_QLEN, 2, 8, _BPW), jnp.float32),
            pltpu.SemaphoreType.DMA,
            pltpu.SemaphoreType.DMA,
        ],
        compiler_params=pltpu.CompilerParams(
            use_tc_tiling_on_sc=True, needs_layout_passes=False
        ),
    )
    def body(qp_hbm, t8_hbm, out_hbm, idx_v, g_v, rows_v, out_v, sem0, sem1):
        wid = lax.axis_index("s") * _NC + lax.axis_index("c")
        base = wid * _BPW
        sems = (sem0, sem1)
        iota = lax.iota(jnp.int32, 16)

        for h in range(3):
            pltpu.sync_copy(
                qp_hbm.at[pl.ds(h * 8, 8), pl.ds(base, _BPW)],
                idx_v.at[pl.ds(h * 8, 8), :],
            )

        def fire(q, buf):
            for k in range(8):
                idx16 = idx_v[q, pl.ds(k * 16, 16)]
                g_v[buf, pl.ds(k * 16, 16)] = lax.shift_right_logical(idx16, 3)
            return pltpu.async_copy(
                t8_hbm.at[g_v.at[buf]], rows_v.at[buf], sems[buf]
            )

        def wait_chunk(buf):
            pltpu.make_async_copy(
                t8_hbm.at[pl.ds(0, _BPW), :], rows_v.at[buf], sems[buf]
            ).wait()

        def extract(q, buf):
            for k in range(8):
                idx16 = idx_v[q, pl.ds(k * 16, 16)]
                colbase = lax.shift_left(
                    lax.bitwise_and(idx16, jnp.int32(7)), 4
                )
                rowids = iota + jnp.int32(k * 16)
                for t in range(16):
                    vals = plsc.load_gather(
                        rows_v.at[buf], [rowids, colbase + jnp.int32(t)]
                    )
                    out_v[q, t // 8, t % 8, pl.ds(k * 16, 16)] = vals

        fire(0, 0)

        @pl.loop(0, _QLEN, step=2)
        def _(q):
            fire(q + 1, 1)
            wait_chunk(0)
            extract(q, 0)

            @pl.when(q + 2 < _QLEN)
            def _():
                fire(q + 2, 0)

            wait_chunk(1)
            extract(q + 1, 1)

        pltpu.sync_copy(out_v, out_hbm.at[:, :, pl.ds(wid * 8, 8), :])

    return body(query_pad, table8)


def kernel(posdoc, query, query_idf, table):
    qpad = jnp.pad(query.T, ((0, 4), (0, 0)))
    table8 = _repack(table.T)
    out5 = _gather(qpad, table8)
    out = out5.reshape(_QLEN, 2, _NW, 8, _BPW)
    out = out.transpose(2, 4, 0, 1, 3)
    return out.reshape(_BATCH, _QLEN, _DIM)


# speedup vs baseline: 1.5010x; 1.5010x over previous
import functools

import jax
import jax.numpy as jnp
from jax import lax
from jax.experimental import pallas as pl
from jax.experimental.pallas import tpu as pltpu
from jax.experimental.pallas import tpu_sc as plsc

_BATCH = 4096
_QLEN = 20
_DIM = 16
_VOCAB = 1000000
_NC = 2
_NS = 16
_NW = _NC * _NS
_BPW = _BATCH // _NW
_G = _VOCAB // 8
_KB = 8192
_KG = 123

_mesh = plsc.VectorSubcoreMesh(core_axis_name="c", subcore_axis_name="s")


_TC = 7813
_CH = 8
_NCHK = 976
_GP = 125008


@jax.jit
def _repack(table_t):
    @functools.partial(
        pl.kernel,
        mesh=_mesh,
        out_type=jax.ShapeDtypeStruct((_GP, 128), jnp.float32),
        scratch_types=[
            pltpu.VMEM((2, _DIM, 128 * _CH), jnp.float32),
            pltpu.VMEM((2, 16 * _CH, 128), jnp.float32),
            pltpu.SemaphoreType.DMA,
            pltpu.SemaphoreType.DMA,
            pltpu.SemaphoreType.DMA,
            pltpu.SemaphoreType.DMA,
        ],
        compiler_params=pltpu.CompilerParams(
            use_tc_tiling_on_sc=True, needs_layout_passes=False
        ),
    )
    def body(tt_hbm, out_hbm, tin, tout, si0, si1, so0, so1):
        wid = lax.axis_index("s") * _NC + lax.axis_index("c")
        sis = (si0, si1)
        sos = (so0, so1)
        iota = lax.iota(jnp.int32, 16)

        def fire_in(j, buf):
            cid = wid + jnp.int32(32) * j
            pltpu.async_copy(
                tt_hbm.at[:, pl.ds(cid * (128 * _CH), 128 * _CH)],
                tin.at[buf],
                sis[buf],
            )

        def wait_in(buf):
            pltpu.make_async_copy(
                tt_hbm.at[:, pl.ds(0, 128 * _CH)], tin.at[buf], sis[buf]
            ).wait()

        def wait_out(buf):
            pltpu.make_async_copy(
                tt_hbm.at[:, pl.ds(0, 128 * _CH)], tout.at[buf], sos[buf]
            ).wait()

        def compute_store(j, buf):
            @plsc.parallel_loop(0, _CH)
            def _(t):
                for g_in in range(16):
                    for a in range(8):
                        col = jnp.full(
                            (16,), t * 128 + jnp.int32(8 * g_in + a), jnp.int32
                        )
                        src = plsc.load_gather(tin.at[buf], [iota, col])
                        tout[buf, 16 * t + g_in, pl.ds(a * 16, 16)] = src

            cid = wid + jnp.int32(32) * j
            pltpu.async_copy(
                tout.at[buf], out_hbm.at[pl.ds(cid * (16 * _CH), 16 * _CH), :],
                sos[buf],
            )

        fire_in(jnp.int32(0), 0)

        @pl.loop(0, 32, step=2)
        def _(j):
            for b in range(2):
                jj = j + b

                @pl.when(wid + 32 * jj < _NCHK)
                def _(jj=jj, b=b):
                    @pl.when(wid + 32 * (jj + 1) < _NCHK)
                    def _():
                        fire_in(jj + 1, 1 - b)

                    wait_in(b)

                    @pl.when(jj >= 2)
                    def _():
                        wait_out(b)

                    compute_store(jj, b)

        @pl.when(wid < _TC - _CH * _NCHK)
        def _():
            tc = jnp.int32(_CH * _NCHK) + wid
            pltpu.async_copy(
                tt_hbm.at[:, pl.ds(tc * 128, 128)],
                tin.at[0, :, pl.ds(0, 128)],
                sis[0],
            )
            pltpu.make_async_copy(
                tt_hbm.at[:, pl.ds(0, 128)],
                tin.at[0, :, pl.ds(0, 128)],
                sis[0],
            ).wait()
            wait_out(0)
            for g_in in range(16):
                for a in range(8):
                    col = jnp.full((16,), 8 * g_in + a, jnp.int32)
                    src = plsc.load_gather(tin.at[0], [iota, col])
                    tout[0, g_in, pl.ds(a * 16, 16)] = src
            pltpu.sync_copy(
                tout.at[0, pl.ds(0, 16), :],
                out_hbm.at[pl.ds(tc * 16, 16), :],
            )

        @pl.when(wid >= _TC - _CH * _NCHK)
        def _():
            wait_out(0)

        wait_out(1)

    return body(table_t)


@jax.jit
def _gather(query_pad, table8):
    @functools.partial(
        pl.kernel,
        mesh=_mesh,
        out_type=jax.ShapeDtypeStruct((_QLEN, 2, 8 * _NW, _BPW), jnp.float32),
        scratch_types=[
            pltpu.VMEM((24, _BPW), jnp.int32),
            pltpu.VMEM((2, _BPW), jnp.int32),
            pltpu.VMEM((2, _BPW, 128), jnp.float32),
            pltpu.VMEM((_QLEN, 2, 8, _BPW), jnp.float32),
            pltpu.SemaphoreType.DMA,
            pltpu.SemaphoreType.DMA,
        ],
        compiler_params=pltpu.CompilerParams(
            use_tc_tiling_on_sc=True, needs_layout_passes=False
        ),
    )
    def body(qp_hbm, t8_hbm, out_hbm, idx_v, g_v, rows_v, out_v, sem0, sem1):
        wid = lax.axis_index("s") * _NC + lax.axis_index("c")
        base = wid * _BPW
        sems = (sem0, sem1)
        iota = lax.iota(jnp.int32, 16)

        for h in range(3):
            pltpu.sync_copy(
                qp_hbm.at[pl.ds(h * 8, 8), pl.ds(base, _BPW)],
                idx_v.at[pl.ds(h * 8, 8), :],
            )

        def fire(q, buf):
            for k in range(8):
                idx16 = idx_v[q, pl.ds(k * 16, 16)]
                g_v[buf, pl.ds(k * 16, 16)] = lax.shift_right_logical(idx16, 3)
            return pltpu.async_copy(
                t8_hbm.at[g_v.at[buf]], rows_v.at[buf], sems[buf]
            )

        def wait_chunk(buf):
            pltpu.make_async_copy(
                t8_hbm.at[pl.ds(0, _BPW), :], rows_v.at[buf], sems[buf]
            ).wait()

        def extract(q, buf):
            for k in range(8):
                idx16 = idx_v[q, pl.ds(k * 16, 16)]
                colbase = lax.shift_left(
                    lax.bitwise_and(idx16, jnp.int32(7)), 4
                )
                rowids = iota + jnp.int32(k * 16)
                for t in range(16):
                    vals = plsc.load_gather(
                        rows_v.at[buf], [rowids, colbase + jnp.int32(t)]
                    )
                    out_v[q, t // 8, t % 8, pl.ds(k * 16, 16)] = vals

        fire(0, 0)

        @pl.loop(0, _QLEN, step=2)
        def _(q):
            fire(q + 1, 1)
            wait_chunk(0)
            extract(q, 0)

            @pl.when(q + 2 < _QLEN)
            def _():
                fire(q + 2, 0)

            wait_chunk(1)
            extract(q + 1, 1)

        pltpu.sync_copy(out_v, out_hbm.at[:, :, pl.ds(wid * 8, 8), :])

    return body(query_pad, table8)


def kernel(posdoc, query, query_idf, table):
    qpad = jnp.pad(query.T, ((0, 4), (0, 0)))
    table8 = _repack(table.T)
    out5 = _gather(qpad, table8)
    out = out5.reshape(_QLEN, 2, _NW, 8, _BPW)
    out = out.transpose(2, 4, 0, 1, 3)
    return out.reshape(_BATCH, _QLEN, _DIM)


# speedup vs baseline: 1.5455x; 1.0296x over previous
import functools

import jax
import jax.numpy as jnp
from jax import lax
from jax.experimental import pallas as pl
from jax.experimental.pallas import tpu as pltpu
from jax.experimental.pallas import tpu_sc as plsc

_BATCH = 4096
_QLEN = 20
_DIM = 16
_VOCAB = 1000000
_NC = 2
_NS = 16
_NW = _NC * _NS
_BPW = _BATCH // _NW
_G = _VOCAB // 8
_KB = 8192
_KG = 123

_mesh = plsc.VectorSubcoreMesh(core_axis_name="c", subcore_axis_name="s")


def _repack_body(x_ref, o_ref):
    x = x_ref[...]
    y3 = x.T.reshape(_KB // 8, 8, 16)
    for a in range(8):
        o_ref[:, a * 16:(a + 1) * 16] = y3[:, a, :]


@jax.jit
def _repack(table_t):
    return pl.pallas_call(
        _repack_body,
        grid=(_KG,),
        in_specs=[pl.BlockSpec((_DIM, _KB), lambda i: (0, i))],
        out_specs=pl.BlockSpec((_KB // 8, 128), lambda i: (i, 0)),
        out_shape=jax.ShapeDtypeStruct((_G, 128), jnp.float32),
    )(table_t)


@jax.jit
def _gather(query_pad, table8):
    @functools.partial(
        pl.kernel,
        mesh=_mesh,
        out_type=jax.ShapeDtypeStruct((_QLEN, 2, 8 * _NW, _BPW), jnp.float32),
        scratch_types=[
            pltpu.VMEM((24, _BPW), jnp.int32),
            pltpu.VMEM((2, _BPW), jnp.int32),
            pltpu.VMEM((2, _BPW, 128), jnp.float32),
            pltpu.VMEM((_QLEN, 2, 8, _BPW), jnp.float32),
            pltpu.SemaphoreType.DMA,
            pltpu.SemaphoreType.DMA,
        ],
        compiler_params=pltpu.CompilerParams(
            use_tc_tiling_on_sc=True, needs_layout_passes=False
        ),
    )
    def body(qp_hbm, t8_hbm, out_hbm, idx_v, g_v, rows_v, out_v, sem0, sem1):
        wid = lax.axis_index("s") * _NC + lax.axis_index("c")
        base = wid * _BPW
        sems = (sem0, sem1)
        iota = lax.iota(jnp.int32, 16)

        for h in range(3):
            pltpu.sync_copy(
                qp_hbm.at[pl.ds(h * 8, 8), pl.ds(base, _BPW)],
                idx_v.at[pl.ds(h * 8, 8), :],
            )

        def fire(q, buf):
            for k in range(8):
                idx16 = idx_v[q, pl.ds(k * 16, 16)]
                g_v[buf, pl.ds(k * 16, 16)] = lax.shift_right_logical(idx16, 3)
            return pltpu.async_copy(
                t8_hbm.at[g_v.at[buf]], rows_v.at[buf], sems[buf]
            )

        def wait_chunk(buf):
            pltpu.make_async_copy(
                t8_hbm.at[pl.ds(0, _BPW), :], rows_v.at[buf], sems[buf]
            ).wait()

        def extract(q, buf):
            for k in range(8):
                idx16 = idx_v[q, pl.ds(k * 16, 16)]
                colbase = lax.shift_left(
                    lax.bitwise_and(idx16, jnp.int32(7)), 4
                )
                rowids = iota + jnp.int32(k * 16)
                for t in range(16):
                    vals = plsc.load_gather(
                        rows_v.at[buf], [rowids, colbase + jnp.int32(t)]
                    )
                    out_v[q, t // 8, t % 8, pl.ds(k * 16, 16)] = vals

        fire(0, 0)

        @pl.loop(0, _QLEN, step=2)
        def _(q):
            fire(q + 1, 1)
            wait_chunk(0)
            extract(q, 0)

            @pl.when(q + 2 < _QLEN)
            def _():
                fire(q + 2, 0)

            wait_chunk(1)
            extract(q + 1, 1)

        pltpu.sync_copy(out_v, out_hbm.at[:, :, pl.ds(wid * 8, 8), :])

    return body(query_pad, table8)


def kernel(posdoc, query, query_idf, table):
    qpad = jnp.pad(query.T, ((0, 4), (0, 0)))
    table8 = _repack(table.T)
    out5 = _gather(qpad, table8)
    out = out5.reshape(_QLEN, 2, _NW, 8, _BPW)
    out = out.transpose(2, 4, 0, 1, 3)
    return out.reshape(_BATCH, _QLEN, _DIM)


# speedup vs baseline: 1.5670x; 1.0139x over previous
import functools

import jax
import jax.numpy as jnp
from jax import lax
from jax.experimental import pallas as pl
from jax.experimental.pallas import tpu as pltpu
from jax.experimental.pallas import tpu_sc as plsc

_BATCH = 4096
_QLEN = 20
_DIM = 16
_VOCAB = 1000000
_NC = 2
_NS = 16
_NW = _NC * _NS
_BPW = _BATCH // _NW
_G = _VOCAB // 8
_KB = 16384
_KG = 62

_mesh = plsc.VectorSubcoreMesh(core_axis_name="c", subcore_axis_name="s")


def _repack_body(x_ref, o_ref):
    x = x_ref[...]
    y3 = x.T.reshape(_KB // 8, 8, 16)
    for a in range(8):
        o_ref[:, a * 16:(a + 1) * 16] = y3[:, a, :]


@jax.jit
def _repack(table_t):
    return pl.pallas_call(
        _repack_body,
        grid=(_KG,),
        in_specs=[pl.BlockSpec((_DIM, _KB), lambda i: (0, i))],
        out_specs=pl.BlockSpec((_KB // 8, 128), lambda i: (i, 0)),
        out_shape=jax.ShapeDtypeStruct((_G, 128), jnp.float32),
    )(table_t)


@jax.jit
def _gather(query_pad, table8):
    @functools.partial(
        pl.kernel,
        mesh=_mesh,
        out_type=jax.ShapeDtypeStruct((_QLEN, 2, 8 * _NW, _BPW), jnp.float32),
        scratch_types=[
            pltpu.VMEM((24, _BPW), jnp.int32),
            pltpu.VMEM((2, _BPW), jnp.int32),
            pltpu.VMEM((2, _BPW, 128), jnp.float32),
            pltpu.VMEM((_QLEN, 2, 8, _BPW), jnp.float32),
            pltpu.SemaphoreType.DMA,
            pltpu.SemaphoreType.DMA,
        ],
        compiler_params=pltpu.CompilerParams(
            use_tc_tiling_on_sc=True, needs_layout_passes=False
        ),
    )
    def body(qp_hbm, t8_hbm, out_hbm, idx_v, g_v, rows_v, out_v, sem0, sem1):
        wid = lax.axis_index("s") * _NC + lax.axis_index("c")
        base = wid * _BPW
        sems = (sem0, sem1)
        iota = lax.iota(jnp.int32, 16)

        for h in range(3):
            pltpu.sync_copy(
                qp_hbm.at[pl.ds(h * 8, 8), pl.ds(base, _BPW)],
                idx_v.at[pl.ds(h * 8, 8), :],
            )

        def fire(q, buf):
            for k in range(8):
                idx16 = idx_v[q, pl.ds(k * 16, 16)]
                g_v[buf, pl.ds(k * 16, 16)] = lax.shift_right_logical(idx16, 3)
            return pltpu.async_copy(
                t8_hbm.at[g_v.at[buf]], rows_v.at[buf], sems[buf]
            )

        def wait_chunk(buf):
            pltpu.make_async_copy(
                t8_hbm.at[pl.ds(0, _BPW), :], rows_v.at[buf], sems[buf]
            ).wait()

        def extract(q, buf):
            for k in range(8):
                idx16 = idx_v[q, pl.ds(k * 16, 16)]
                colbase = lax.shift_left(
                    lax.bitwise_and(idx16, jnp.int32(7)), 4
                )
                rowids = iota + jnp.int32(k * 16)
                for t in range(16):
                    vals = plsc.load_gather(
                        rows_v.at[buf], [rowids, colbase + jnp.int32(t)]
                    )
                    out_v[q, t // 8, t % 8, pl.ds(k * 16, 16)] = vals

        fire(0, 0)

        @pl.loop(0, _QLEN, step=2)
        def _(q):
            fire(q + 1, 1)
            wait_chunk(0)
            extract(q, 0)

            @pl.when(q + 2 < _QLEN)
            def _():
                fire(q + 2, 0)

            wait_chunk(1)
            extract(q + 1, 1)

        pltpu.sync_copy(out_v, out_hbm.at[:, :, pl.ds(wid * 8, 8), :])

    return body(query_pad, table8)


def kernel(posdoc, query, query_idf, table):
    qpad = jnp.pad(query.T, ((0, 4), (0, 0)))
    table8 = _repack(table.T)
    out5 = _gather(qpad, table8)
    out = out5.reshape(_QLEN, 2, _NW, 8, _BPW)
    out = out.transpose(2, 4, 0, 1, 3)
    return out.reshape(_BATCH, _QLEN, _DIM)
